# baseline (speedup 1.0000x reference)
import functools

import jax
import jax.numpy as jnp
import numpy as np
from jax import lax
from jax.experimental import pallas as pl
from jax.experimental.pallas import tpu as pltpu
from jax.experimental.pallas import tpu_sc as plsc

N = 10000
E = 320000
D = 128
NC = 2
NS = 16
NW = NC * NS
LANES = 16
EDGES_PER_TILE = E // NW
CHUNK = 80
NCHUNK = EDGES_PER_TILE // CHUNK
ROWS_PER_TILE = N // NS

_COL_PERM = np.concatenate(
    [np.concatenate([np.arange(q * 32, (q + 1) * 32, 2),
                     np.arange(q * 32 + 1, (q + 1) * 32, 2)])
     for q in range(D // 32)])


def _lane_broadcast(v16, e):
    idx = jnp.full((LANES, 1), e, jnp.int32)
    dn = lax.GatherDimensionNumbers(
        offset_dims=(), collapsed_slice_dims=(0,), start_index_map=(0,))
    return lax.gather(v16, idx, dn, (1,),
                      mode=lax.GatherScatterMode.PROMISE_IN_BOUNDS)


def _spmm_partials(emb_bf, src, dst, vals):
    mesh = plsc.VectorSubcoreMesh(
        core_axis_name="c", subcore_axis_name="s", num_cores=NC,
        num_subcores=NS)

    @functools.partial(
        pl.kernel,
        out_type=jax.ShapeDtypeStruct((NC, N, D), jnp.float32),
        mesh=mesh,
        compiler_params=pltpu.CompilerParams(use_tc_tiling_on_sc=False, needs_layout_passes=False),
        scratch_types=[
            pltpu.VMEM_SHARED((N, D), jnp.float32),
            pltpu.VMEM((NCHUNK, CHUNK), jnp.int32),
            pltpu.VMEM((NCHUNK, CHUNK), jnp.int32),
            pltpu.VMEM((EDGES_PER_TILE,), jnp.float32),
            pltpu.VMEM((CHUNK, D), jnp.bfloat16),
            pltpu.VMEM((CHUNK, D), jnp.bfloat16),
            pltpu.VMEM((CHUNK, D), jnp.float32),
            pltpu.SemaphoreType.DMA,
            pltpu.SemaphoreType.DMA,
        ],
    )
    def spmm(emb_hbm, src_hbm, dst_hbm, val_hbm, out_hbm,
             acc_sh, src_v, dst_v, val_v, g0, g1, sbuf, sem0, sem1):
        c = lax.axis_index("c")
        s = lax.axis_index("s")
        wid = s * NC + c

        pltpu.sync_copy(src_hbm.at[wid], src_v)
        pltpu.sync_copy(dst_hbm.at[wid], dst_v)
        pltpu.sync_copy(val_hbm.at[wid], val_v)

        zeros16 = jnp.zeros((LANES,), jnp.float32)

        def zrow(r, carry):
            for j in range(D // LANES):
                sbuf[r, pl.ds(j * LANES, LANES)] = zeros16
            return carry

        lax.fori_loop(0, CHUNK, zrow, 0)
        base = s * ROWS_PER_TILE
        for t in range(ROWS_PER_TILE // CHUNK):
            pltpu.sync_copy(sbuf, acc_sh.at[pl.ds(base + t * CHUNK, CHUNK)])
        rem = ROWS_PER_TILE % CHUNK
        if rem:
            pltpu.sync_copy(
                sbuf.at[pl.ds(0, rem)],
                acc_sh.at[pl.ds(base + ROWS_PER_TILE - rem, rem)])
        plsc.subcore_barrier()

        def start_gather(ci, gbuf, sem):
            pltpu.async_copy(emb_hbm.at[src_v.at[ci]], gbuf, sem)

        def wait_gather(gbuf, sem):
            pltpu.make_async_copy(emb_hbm.at[src_v.at[0]], gbuf, sem).wait()

        mask_hi = jnp.full((LANES,), -65536, jnp.int32)

        def scale_widen(gbuf, ci):
            def group(g, gcarry):
                a16 = val_v[pl.ds(ci * CHUNK + g * LANES, LANES)]
                for e in range(LANES):
                    ae = _lane_broadcast(a16, e)
                    r = g * LANES + e
                    for q in range(D // 32):
                        w = plsc.bitcast(gbuf[r, pl.ds(q * 32, 32)],
                                         jnp.int32)
                        even = plsc.bitcast(
                            lax.shift_left(w, 16), jnp.float32)
                        odd = plsc.bitcast(
                            jnp.bitwise_and(w, mask_hi), jnp.float32)
                        sbuf[r, pl.ds(q * 32, LANES)] = even * ae
                        sbuf[r, pl.ds(q * 32 + LANES, LANES)] = odd * ae
                return gcarry

            lax.fori_loop(0, CHUNK // LANES, group, 0)

        def scatter(ci):
            pltpu.sync_copy(sbuf, acc_sh.at[dst_v.at[ci]], add=True)

        start_gather(0, g0, sem0)

        def chunk_pair(t, carry):
            i0 = 2 * t
            start_gather(i0 + 1, g1, sem1)
            wait_gather(g0, sem0)
            scale_widen(g0, i0)
            start_gather(i0 + 2, g0, sem0)
            scatter(i0)
            wait_gather(g1, sem1)
            scale_widen(g1, i0 + 1)
            scatter(i0 + 1)
            return carry

        lax.fori_loop(0, (NCHUNK - 1) // 2, chunk_pair, 0)
        wait_gather(g0, sem0)
        scale_widen(g0, NCHUNK - 1)
        scatter(NCHUNK - 1)
        plsc.subcore_barrier()

        pltpu.sync_copy(
            acc_sh.at[pl.ds(s * ROWS_PER_TILE, ROWS_PER_TILE)],
            out_hbm.at[c, pl.ds(s * ROWS_PER_TILE, ROWS_PER_TILE)])

    return spmm(emb_bf, src, dst, vals)


_BLK = 1000


_P_SPEC_A = pl.BlockSpec((1, _BLK, D), lambda i: (0, i, 0))
_P_SPEC_B = pl.BlockSpec((1, _BLK, D), lambda i: (1, i, 0))
_W_SPEC = pl.BlockSpec((D, D), lambda i: (0, 0))
_ROW_SPEC = pl.BlockSpec((_BLK, D), lambda i: (i, 0))


def _dense_body(pa_ref, pb_ref, w_ref, o_ref, obf_ref):
    x = pa_ref[0] + pb_ref[0]
    y = jnp.dot(x, w_ref[...], preferred_element_type=jnp.float32)
    e = 1.0 / (1.0 + jnp.exp(-y))
    o_ref[...] = e
    obf_ref[...] = e.astype(jnp.bfloat16)


def _dense(partials, W_perm):
    return pl.pallas_call(
        _dense_body,
        grid=(N // _BLK,),
        in_specs=[_P_SPEC_A, _P_SPEC_B, _W_SPEC],
        out_specs=[_ROW_SPEC, _ROW_SPEC],
        out_shape=[jax.ShapeDtypeStruct((N, D), jnp.float32),
                   jax.ShapeDtypeStruct((N, D), jnp.bfloat16)],
    )(partials, partials, W_perm)


def _dense_final_body(pa_ref, pb_ref, w_ref, x0_ref, x1_ref, x2_ref,
                      o3_ref, om_ref):
    x = pa_ref[0] + pb_ref[0]
    y = jnp.dot(x, w_ref[...], preferred_element_type=jnp.float32)
    e3 = 1.0 / (1.0 + jnp.exp(-y))
    o3_ref[...] = e3
    om_ref[...] = (x0_ref[...] + x1_ref[...] + x2_ref[...] + e3) * 0.25


def _dense_final(partials, W_perm, e0, e1, e2):
    return pl.pallas_call(
        _dense_final_body,
        grid=(N // _BLK,),
        in_specs=[_P_SPEC_A, _P_SPEC_B, _W_SPEC,
                  _ROW_SPEC, _ROW_SPEC, _ROW_SPEC],
        out_specs=[_ROW_SPEC, _ROW_SPEC],
        out_shape=[jax.ShapeDtypeStruct((N, D), jnp.float32),
                   jax.ShapeDtypeStruct((N, D), jnp.float32)],
    )(partials, partials, W_perm, e0, e1, e2)


def kernel(ItemAndUserEmebddings, edge_index, A_values, W0, W1, W2):
    x = ItemAndUserEmebddings
    src = edge_index[0].reshape(NW, NCHUNK, CHUNK)
    dst = edge_index[1].reshape(NW, NCHUNK, CHUNK)
    A_values = A_values.reshape(NW, EDGES_PER_TILE)
    perm = jnp.asarray(_COL_PERM)
    W0p, W1p, W2p = W0[perm, :], W1[perm, :], W2[perm, :]

    p = _spmm_partials(x.astype(jnp.bfloat16), src, dst, A_values)
    e1, e1_bf = _dense(p, W0p)
    p = _spmm_partials(e1_bf, src, dst, A_values)
    e2, e2_bf = _dense(p, W1p)
    p = _spmm_partials(e2_bf, src, dst, A_values)
    e3, mean = _dense_final(p, W2p, x, e1, e2)
    return (mean, x, e1, e2, e3)


# speedup vs baseline: 2.2991x; 2.2991x over previous
import functools

import jax
import jax.numpy as jnp
from jax import lax
from jax.experimental import pallas as pl
from jax.experimental.pallas import tpu as pltpu
from jax.experimental.pallas import tpu_sc as plsc

N = 10000
E = 320000
D = 128
NC = 2
NS = 16
NW = NC * NS
LANES = 16
EDGES_PER_TILE = E // NW
CHUNK = 100
NCHUNK = EDGES_PER_TILE // CHUNK
NBUF = 3
ROWS_PER_TILE = N // NS
FULL_GROUPS = CHUNK // LANES
TAIL = CHUNK - FULL_GROUPS * LANES


def _lane_broadcast(v16, e):
    idx = jnp.full((LANES, 1), e, jnp.int32)
    dn = lax.GatherDimensionNumbers(
        offset_dims=(), collapsed_slice_dims=(0,), start_index_map=(0,))
    return lax.gather(v16, idx, dn, (1,),
                      mode=lax.GatherScatterMode.PROMISE_IN_BOUNDS)


def _spmm_partials(emb, src, dst, vals):
    mesh = plsc.VectorSubcoreMesh(
        core_axis_name="c", subcore_axis_name="s", num_cores=NC,
        num_subcores=NS)

    @functools.partial(
        pl.kernel,
        out_type=jax.ShapeDtypeStruct((NC, N, D), jnp.float32),
        mesh=mesh,
        compiler_params=pltpu.CompilerParams(use_tc_tiling_on_sc=False),
        scratch_types=[
            pltpu.VMEM_SHARED((N, D), jnp.float32),
            pltpu.VMEM((NCHUNK, CHUNK), jnp.int32),
            [pltpu.VMEM((CHUNK, D), jnp.float32)] * NBUF,
            [pltpu.VMEM((CHUNK,), jnp.int32)] * NBUF,
            [pltpu.VMEM((CHUNK,), jnp.float32)] * NBUF,
            [pltpu.SemaphoreType.DMA] * NBUF,
            [pltpu.SemaphoreType.DMA] * NBUF,
            pltpu.SemaphoreType.DMA,
        ],
    )
    def spmm(emb_hbm, src_hbm, dst_hbm, val_hbm, out_hbm,
             acc_sh, src_v, gbufs, dbufs, vbufs, gsems, dsems, zsem):
        c = lax.axis_index("c")
        s = lax.axis_index("s")
        wid = s * NC + c

        pltpu.sync_copy(src_hbm.at[wid], src_v)

        zeros16 = jnp.zeros((LANES,), jnp.float32)
        zb = gbufs[0]

        def zrow(r, carry):
            for j in range(D // LANES):
                zb[r, pl.ds(j * LANES, LANES)] = zeros16
            return carry

        lax.fori_loop(0, CHUNK, zrow, 0)
        base = s * ROWS_PER_TILE
        nfull = ROWS_PER_TILE // CHUNK
        rem = ROWS_PER_TILE % CHUNK
        for t in range(nfull):
            pltpu.async_copy(zb, acc_sh.at[pl.ds(base + t * CHUNK, CHUNK)],
                             zsem)
        if rem:
            pltpu.async_copy(
                zb.at[pl.ds(0, rem)],
                acc_sh.at[pl.ds(base + ROWS_PER_TILE - rem, rem)], zsem)
        for t in range(nfull):
            pltpu.make_async_copy(
                zb, acc_sh.at[pl.ds(base + t * CHUNK, CHUNK)], zsem).wait()
        if rem:
            pltpu.make_async_copy(
                zb.at[pl.ds(0, rem)],
                acc_sh.at[pl.ds(base + ROWS_PER_TILE - rem, rem)],
                zsem).wait()
        plsc.subcore_barrier()

        def start_chunk(ci, k):
            pltpu.async_copy(emb_hbm.at[src_v.at[ci]], gbufs[k], gsems[k])
            pltpu.async_copy(dst_hbm.at[wid, ci], dbufs[k], dsems[k])
            pltpu.async_copy(val_hbm.at[wid, ci], vbufs[k], dsems[k])

        def wait_chunk(k):
            pltpu.make_async_copy(
                emb_hbm.at[src_v.at[0]], gbufs[k], gsems[k]).wait()
            pltpu.make_async_copy(
                dst_hbm.at[wid, 0], dbufs[k], dsems[k]).wait()
            pltpu.make_async_copy(
                val_hbm.at[wid, 0], vbufs[k], dsems[k]).wait()

        def scale(k):
            gb, vb = gbufs[k], vbufs[k]

            def one(e, a16, r0):
                ae = _lane_broadcast(a16, e)
                for j in range(D // LANES):
                    sl = pl.ds(j * LANES, LANES)
                    gb[r0 + e, sl] = gb[r0 + e, sl] * ae

            def group(g, gcarry):
                a16 = vb[pl.ds(g * LANES, LANES)]
                for e in range(LANES):
                    one(e, a16, g * LANES)
                return gcarry

            lax.fori_loop(0, FULL_GROUPS, group, 0)
            if TAIL:
                a16 = vb[pl.ds(CHUNK - LANES, LANES)]
                for e in range(LANES - TAIL, LANES):
                    one(e, a16, CHUNK - LANES)

        def scatter(k):
            pltpu.sync_copy(gbufs[k], acc_sh.at[dbufs[k]], add=True)

        for k in range(NBUF):
            start_chunk(k, k)

        def ring(t, carry):
            i0 = NBUF * t
            for k in range(NBUF):
                i = i0 + k
                wait_chunk(k)
                scale(k)
                scatter(k)

                @pl.when(i + NBUF < NCHUNK)
                def _():
                    start_chunk(i + NBUF, k)
            return carry

        lax.fori_loop(0, NCHUNK // NBUF, ring, 0)
        wait_chunk(0)
        scale(0)
        scatter(0)
        plsc.subcore_barrier()

        pltpu.sync_copy(
            acc_sh.at[pl.ds(s * ROWS_PER_TILE, ROWS_PER_TILE)],
            out_hbm.at[c, pl.ds(s * ROWS_PER_TILE, ROWS_PER_TILE)])

    return spmm(emb, src, dst, vals)


_BLK = 1000


_P_SPEC_A = pl.BlockSpec((1, _BLK, D), lambda i: (0, i, 0))
_P_SPEC_B = pl.BlockSpec((1, _BLK, D), lambda i: (1, i, 0))
_W_SPEC = pl.BlockSpec((D, D), lambda i: (0, 0))
_ROW_SPEC = pl.BlockSpec((_BLK, D), lambda i: (i, 0))


def _dense_body(pa_ref, pb_ref, w_ref, o_ref):
    x = pa_ref[0] + pb_ref[0]
    y = jnp.dot(x, w_ref[...], preferred_element_type=jnp.float32)
    o_ref[...] = 1.0 / (1.0 + jnp.exp(-y))


def _dense(partials, W):
    return pl.pallas_call(
        _dense_body,
        grid=(N // _BLK,),
        in_specs=[_P_SPEC_A, _P_SPEC_B, _W_SPEC],
        out_specs=_ROW_SPEC,
        out_shape=jax.ShapeDtypeStruct((N, D), jnp.float32),
    )(partials, partials, W)


def _dense_final_body(pa_ref, pb_ref, w_ref, x0_ref, x1_ref, x2_ref,
                      o3_ref, om_ref):
    x = pa_ref[0] + pb_ref[0]
    y = jnp.dot(x, w_ref[...], preferred_element_type=jnp.float32)
    e3 = 1.0 / (1.0 + jnp.exp(-y))
    o3_ref[...] = e3
    om_ref[...] = (x0_ref[...] + x1_ref[...] + x2_ref[...] + e3) * 0.25


def _dense_final(partials, W, e0, e1, e2):
    return pl.pallas_call(
        _dense_final_body,
        grid=(N // _BLK,),
        in_specs=[_P_SPEC_A, _P_SPEC_B, _W_SPEC,
                  _ROW_SPEC, _ROW_SPEC, _ROW_SPEC],
        out_specs=[_ROW_SPEC, _ROW_SPEC],
        out_shape=[jax.ShapeDtypeStruct((N, D), jnp.float32),
                   jax.ShapeDtypeStruct((N, D), jnp.float32)],
    )(partials, partials, W, e0, e1, e2)


def kernel(ItemAndUserEmebddings, edge_index, A_values, W0, W1, W2):
    x = ItemAndUserEmebddings
    src = edge_index[0].reshape(NW, NCHUNK, CHUNK)
    dst = edge_index[1].reshape(NW, NCHUNK, CHUNK)
    A_values = A_values.reshape(NW, NCHUNK, CHUNK)

    p = _spmm_partials(x, src, dst, A_values)
    e1 = _dense(p, W0)
    p = _spmm_partials(e1, src, dst, A_values)
    e2 = _dense(p, W1)
    p = _spmm_partials(e2, src, dst, A_values)
    e3, mean = _dense_final(p, W2, x, e1, e2)
    return (mean, x, e1, e2, e3)


# speedup vs baseline: 2.3731x; 1.0322x over previous
import functools

import jax
import jax.numpy as jnp
from jax import lax
from jax.experimental import pallas as pl
from jax.experimental.pallas import tpu as pltpu
from jax.experimental.pallas import tpu_sc as plsc

N = 10000
E = 320000
D = 128
NC = 2
NS = 16
NW = NC * NS
LANES = 16
EDGES_PER_TILE = E // NW
CHUNK = 100
NCHUNK = EDGES_PER_TILE // CHUNK
NBUF = 3
ROWS_PER_TILE = N // NS
FULL_GROUPS = CHUNK // LANES
TAIL = CHUNK - FULL_GROUPS * LANES


def _lane_broadcast(v16, e):
    idx = jnp.full((LANES, 1), e, jnp.int32)
    dn = lax.GatherDimensionNumbers(
        offset_dims=(), collapsed_slice_dims=(0,), start_index_map=(0,))
    return lax.gather(v16, idx, dn, (1,),
                      mode=lax.GatherScatterMode.PROMISE_IN_BOUNDS)


def _spmm_partials(emb, src, dst, vals):
    mesh = plsc.VectorSubcoreMesh(
        core_axis_name="c", subcore_axis_name="s", num_cores=NC,
        num_subcores=NS)

    @functools.partial(
        pl.kernel,
        out_type=jax.ShapeDtypeStruct((NC, N, D), jnp.float32),
        mesh=mesh,
        compiler_params=pltpu.CompilerParams(use_tc_tiling_on_sc=False),
        scratch_types=[
            pltpu.VMEM_SHARED((N, D), jnp.float32),
            pltpu.VMEM((NCHUNK, CHUNK), jnp.int32),
            [pltpu.VMEM((CHUNK, D), jnp.float32)] * NBUF,
            [pltpu.VMEM((CHUNK,), jnp.int32)] * NBUF,
            [pltpu.VMEM((CHUNK,), jnp.float32)] * NBUF,
            pltpu.VMEM((8, D), jnp.float32),
            [pltpu.SemaphoreType.DMA] * NBUF,
            [pltpu.SemaphoreType.DMA] * NBUF,
            [pltpu.SemaphoreType.DMA] * NBUF,
            pltpu.SemaphoreType.DMA,
        ],
    )
    def spmm(emb_hbm, src_hbm, dst_hbm, val_hbm, out_hbm,
             acc_sh, src_v, gbufs, dbufs, vbufs, zb, gsems, dsems, ssems,
             zsem):
        c = lax.axis_index("c")
        s = lax.axis_index("s")
        wid = s * NC + c

        pltpu.sync_copy(src_hbm.at[wid], src_v)

        def start_chunk(ci, k):
            pltpu.async_copy(emb_hbm.at[src_v.at[ci]], gbufs[k], gsems[k])
            pltpu.async_copy(dst_hbm.at[wid, ci], dbufs[k], dsems[k])
            pltpu.async_copy(val_hbm.at[wid, ci], vbufs[k], dsems[k])

        for k in range(NBUF):
            start_chunk(k, k)

        zeros16 = jnp.zeros((LANES,), jnp.float32)

        def zrow(r, carry):
            for j in range(D // LANES):
                zb[r, pl.ds(j * LANES, LANES)] = zeros16
            return carry

        lax.fori_loop(0, 8, zrow, 0)
        base = s * ROWS_PER_TILE
        nfull = ROWS_PER_TILE // 8
        rem = ROWS_PER_TILE % 8
        for t in range(nfull):
            pltpu.async_copy(zb, acc_sh.at[pl.ds(base + t * 8, 8)], zsem)
        if rem:
            pltpu.async_copy(
                zb.at[pl.ds(0, rem)],
                acc_sh.at[pl.ds(base + ROWS_PER_TILE - rem, rem)], zsem)
        for t in range(nfull):
            pltpu.make_async_copy(
                zb, acc_sh.at[pl.ds(base + t * 8, 8)], zsem).wait()
        if rem:
            pltpu.make_async_copy(
                zb.at[pl.ds(0, rem)],
                acc_sh.at[pl.ds(base + ROWS_PER_TILE - rem, rem)],
                zsem).wait()
        plsc.subcore_barrier()

        def wait_chunk(k):
            pltpu.make_async_copy(
                emb_hbm.at[src_v.at[0]], gbufs[k], gsems[k]).wait()
            pltpu.make_async_copy(
                dst_hbm.at[wid, 0], dbufs[k], dsems[k]).wait()
            pltpu.make_async_copy(
                val_hbm.at[wid, 0], vbufs[k], dsems[k]).wait()

        def scale(k):
            gb, vb = gbufs[k], vbufs[k]

            def one(e, a16, r0):
                ae = _lane_broadcast(a16, e)
                for j in range(D // LANES):
                    sl = pl.ds(j * LANES, LANES)
                    gb[r0 + e, sl] = gb[r0 + e, sl] * ae

            def group(g, gcarry):
                a16 = vb[pl.ds(g * LANES, LANES)]
                for e in range(LANES):
                    one(e, a16, g * LANES)
                return gcarry

            lax.fori_loop(0, FULL_GROUPS, group, 0)
            if TAIL:
                a16 = vb[pl.ds(CHUNK - LANES, LANES)]
                for e in range(LANES - TAIL, LANES):
                    one(e, a16, CHUNK - LANES)

        def start_scatter(k):
            pltpu.make_async_copy(
                gbufs[k], acc_sh.at[dbufs[k]], ssems[k]).start(add=True)

        def wait_scatter(k):
            pltpu.make_async_copy(
                gbufs[k], acc_sh.at[dbufs[k]], ssems[k]).wait()

        def ring(t, carry):
            i0 = NBUF * t
            for k in range(NBUF):
                i = i0 + k
                wait_chunk(k)
                scale(k)
                start_scatter(k)
                kp = (k - 1) % NBUF

                @pl.when(i >= 1)
                def _():
                    wait_scatter(kp)

                @pl.when((i >= 1) & (i - 1 + NBUF < NCHUNK))
                def _():
                    start_chunk(i - 1 + NBUF, kp)
            return carry

        lax.fori_loop(0, NCHUNK // NBUF, ring, 0)
        wait_scatter(2)
        wait_chunk(0)
        scale(0)
        pltpu.sync_copy(gbufs[0], acc_sh.at[dbufs[0]], add=True)
        plsc.subcore_barrier()

        pltpu.sync_copy(
            acc_sh.at[pl.ds(s * ROWS_PER_TILE, ROWS_PER_TILE)],
            out_hbm.at[c, pl.ds(s * ROWS_PER_TILE, ROWS_PER_TILE)])

    return spmm(emb, src, dst, vals)


_BLK = 1000


_P_SPEC_A = pl.BlockSpec((1, _BLK, D), lambda i: (0, i, 0))
_P_SPEC_B = pl.BlockSpec((1, _BLK, D), lambda i: (1, i, 0))
_W_SPEC = pl.BlockSpec((D, D), lambda i: (0, 0))
_ROW_SPEC = pl.BlockSpec((_BLK, D), lambda i: (i, 0))


def _dense_body(pa_ref, pb_ref, w_ref, o_ref):
    x = pa_ref[0] + pb_ref[0]
    y = jnp.dot(x, w_ref[...], preferred_element_type=jnp.float32)
    o_ref[...] = 1.0 / (1.0 + jnp.exp(-y))


def _dense(partials, W):
    return pl.pallas_call(
        _dense_body,
        grid=(N // _BLK,),
        in_specs=[_P_SPEC_A, _P_SPEC_B, _W_SPEC],
        out_specs=_ROW_SPEC,
        out_shape=jax.ShapeDtypeStruct((N, D), jnp.float32),
    )(partials, partials, W)


def _dense_final_body(pa_ref, pb_ref, w_ref, x0_ref, x1_ref, x2_ref,
                      o3_ref, om_ref):
    x = pa_ref[0] + pb_ref[0]
    y = jnp.dot(x, w_ref[...], preferred_element_type=jnp.float32)
    e3 = 1.0 / (1.0 + jnp.exp(-y))
    o3_ref[...] = e3
    om_ref[...] = (x0_ref[...] + x1_ref[...] + x2_ref[...] + e3) * 0.25


def _dense_final(partials, W, e0, e1, e2):
    return pl.pallas_call(
        _dense_final_body,
        grid=(N // _BLK,),
        in_specs=[_P_SPEC_A, _P_SPEC_B, _W_SPEC,
                  _ROW_SPEC, _ROW_SPEC, _ROW_SPEC],
        out_specs=[_ROW_SPEC, _ROW_SPEC],
        out_shape=[jax.ShapeDtypeStruct((N, D), jnp.float32),
                   jax.ShapeDtypeStruct((N, D), jnp.float32)],
    )(partials, partials, W, e0, e1, e2)


def kernel(ItemAndUserEmebddings, edge_index, A_values, W0, W1, W2):
    x = ItemAndUserEmebddings
    src = edge_index[0].reshape(NW, NCHUNK, CHUNK)
    dst = edge_index[1].reshape(NW, NCHUNK, CHUNK)
    A_values = A_values.reshape(NW, NCHUNK, CHUNK)

    p = _spmm_partials(x, src, dst, A_values)
    e1 = _dense(p, W0)
    p = _spmm_partials(e1, src, dst, A_values)
    e2 = _dense(p, W1)
    p = _spmm_partials(e2, src, dst, A_values)
    e3, mean = _dense_final(p, W2, x, e1, e2)
    return (mean, x, e1, e2, e3)


# speedup vs baseline: 2.4876x; 1.0482x over previous
import functools

import jax
import jax.numpy as jnp
from jax import lax
from jax.experimental import pallas as pl
from jax.experimental.pallas import tpu as pltpu
from jax.experimental.pallas import tpu_sc as plsc

N = 10000
E = 320000
D = 128
NC = 2
NS = 16
NW = NC * NS
LANES = 16
EDGES_PER_TILE = E // NW
CHUNK = 80
NCHUNK = EDGES_PER_TILE // CHUNK
NBUF = 4
ROWS_PER_TILE = N // NS
FULL_GROUPS = CHUNK // LANES
TAIL = CHUNK - FULL_GROUPS * LANES


def _lane_broadcast(v16, e):
    idx = jnp.full((LANES, 1), e, jnp.int32)
    dn = lax.GatherDimensionNumbers(
        offset_dims=(), collapsed_slice_dims=(0,), start_index_map=(0,))
    return lax.gather(v16, idx, dn, (1,),
                      mode=lax.GatherScatterMode.PROMISE_IN_BOUNDS)


def _spmm_partials(emb, src, dst, vals):
    mesh = plsc.VectorSubcoreMesh(
        core_axis_name="c", subcore_axis_name="s", num_cores=NC,
        num_subcores=NS)

    @functools.partial(
        pl.kernel,
        out_type=jax.ShapeDtypeStruct((NC, N, D), jnp.float32),
        mesh=mesh,
        compiler_params=pltpu.CompilerParams(use_tc_tiling_on_sc=False),
        scratch_types=[
            pltpu.VMEM_SHARED((N, D), jnp.float32),
            [pltpu.VMEM((CHUNK, D), jnp.float32)] * NBUF,
            [pltpu.VMEM((CHUNK,), jnp.int32)] * NBUF,
            [pltpu.VMEM((CHUNK,), jnp.int32)] * NBUF,
            [pltpu.VMEM((CHUNK,), jnp.float32)] * NBUF,
            [pltpu.SemaphoreType.DMA] * NBUF,
            [pltpu.SemaphoreType.DMA] * NBUF,
            [pltpu.SemaphoreType.DMA] * NBUF,
            [pltpu.SemaphoreType.DMA] * NBUF,
            pltpu.SemaphoreType.DMA,
        ],
    )
    def spmm(emb_hbm, src_hbm, dst_hbm, val_hbm, out_hbm,
             acc_sh, gbufs, sbufs, dbufs, vbufs, gsems, isems, dsems,
             ssems, zsem):
        c = lax.axis_index("c")
        s = lax.axis_index("s")
        wid = s * NC + c

        def start_src(ci, k):
            pltpu.async_copy(src_hbm.at[wid, ci], sbufs[k], isems[k])

        def wait_src(k):
            pltpu.make_async_copy(
                src_hbm.at[wid, 0], sbufs[k], isems[k]).wait()

        def start_dv(ci, k):
            pltpu.async_copy(dst_hbm.at[wid, ci], dbufs[k], dsems[k])
            pltpu.async_copy(val_hbm.at[wid, ci], vbufs[k], dsems[k])

        def start_gather(k):
            pltpu.async_copy(emb_hbm.at[sbufs[k]], gbufs[k], gsems[k])

        def wait_chunk(k):
            pltpu.make_async_copy(
                emb_hbm.at[sbufs[0]], gbufs[k], gsems[k]).wait()
            pltpu.make_async_copy(
                dst_hbm.at[wid, 0], dbufs[k], dsems[k]).wait()
            pltpu.make_async_copy(
                val_hbm.at[wid, 0], vbufs[k], dsems[k]).wait()

        def start_scatter(k):
            pltpu.make_async_copy(
                gbufs[k], acc_sh.at[dbufs[k]], ssems[k]).start(add=True)

        def wait_scatter(k):
            pltpu.make_async_copy(
                gbufs[k], acc_sh.at[dbufs[k]], ssems[k]).wait()

        def scale(k):
            gb, vb = gbufs[k], vbufs[k]

            def one(e, a16, r0):
                ae = _lane_broadcast(a16, e)
                for j in range(D // LANES):
                    sl = pl.ds(j * LANES, LANES)
                    gb[r0 + e, sl] = gb[r0 + e, sl] * ae

            def group(g, gcarry):
                a16 = vb[pl.ds(g * LANES, LANES)]
                for e in range(LANES):
                    one(e, a16, g * LANES)
                return gcarry

            lax.fori_loop(0, CHUNK // LANES, group, 0)

        for k in range(NBUF - 1):
            start_src(k, k)
            start_dv(k, k)
        for k in range(NBUF - 1):
            wait_src(k)
            start_gather(k)

        zeros16 = jnp.zeros((LANES,), jnp.float32)
        zb = gbufs[NBUF - 1]

        def zrow(r, carry):
            for j in range(D // LANES):
                zb[r, pl.ds(j * LANES, LANES)] = zeros16
            return carry

        lax.fori_loop(0, CHUNK, zrow, 0)
        base = s * ROWS_PER_TILE
        nfull = ROWS_PER_TILE // CHUNK
        rem = ROWS_PER_TILE % CHUNK
        for t in range(nfull):
            pltpu.async_copy(zb, acc_sh.at[pl.ds(base + t * CHUNK, CHUNK)],
                             zsem)
        if rem:
            pltpu.async_copy(
                zb.at[pl.ds(0, rem)],
                acc_sh.at[pl.ds(base + ROWS_PER_TILE - rem, rem)], zsem)
        for t in range(nfull):
            pltpu.make_async_copy(
                zb, acc_sh.at[pl.ds(base + t * CHUNK, CHUNK)], zsem).wait()
        if rem:
            pltpu.make_async_copy(
                zb.at[pl.ds(0, rem)],
                acc_sh.at[pl.ds(base + ROWS_PER_TILE - rem, rem)],
                zsem).wait()
        plsc.subcore_barrier()

        def ring(t, carry):
            i0 = NBUF * t
            for k in range(NBUF):
                i = i0 + k
                kp = (k - 1) % NBUF
                wait_chunk(k)

                @pl.when(i + NBUF - 1 < NCHUNK)
                def _():
                    start_src(i + NBUF - 1, kp)

                scale(k)
                start_scatter(k)

                @pl.when(i >= 1)
                def _():
                    wait_scatter(kp)

                @pl.when(i + NBUF - 1 < NCHUNK)
                def _():
                    start_dv(i + NBUF - 1, kp)
                    wait_src(kp)
                    start_gather(kp)
            return carry

        lax.fori_loop(0, NCHUNK // NBUF, ring, 0)
        wait_scatter(NBUF - 1)
        wait_chunk(0)
        scale(0)
        pltpu.sync_copy(gbufs[0], acc_sh.at[dbufs[0]], add=True)
        plsc.subcore_barrier()

        pltpu.sync_copy(
            acc_sh.at[pl.ds(s * ROWS_PER_TILE, ROWS_PER_TILE)],
            out_hbm.at[c, pl.ds(s * ROWS_PER_TILE, ROWS_PER_TILE)])

    return spmm(emb, src, dst, vals)


_BLK = 1000


_P_SPEC_A = pl.BlockSpec((1, _BLK, D), lambda i: (0, i, 0))
_P_SPEC_B = pl.BlockSpec((1, _BLK, D), lambda i: (1, i, 0))
_W_SPEC = pl.BlockSpec((D, D), lambda i: (0, 0))
_ROW_SPEC = pl.BlockSpec((_BLK, D), lambda i: (i, 0))


def _dense_body(pa_ref, pb_ref, w_ref, o_ref):
    x = pa_ref[0] + pb_ref[0]
    y = jnp.dot(x, w_ref[...], preferred_element_type=jnp.float32)
    o_ref[...] = 1.0 / (1.0 + jnp.exp(-y))


def _dense(partials, W):
    return pl.pallas_call(
        _dense_body,
        grid=(N // _BLK,),
        in_specs=[_P_SPEC_A, _P_SPEC_B, _W_SPEC],
        out_specs=_ROW_SPEC,
        out_shape=jax.ShapeDtypeStruct((N, D), jnp.float32),
    )(partials, partials, W)


def _dense_final_body(pa_ref, pb_ref, w_ref, x0_ref, x1_ref, x2_ref,
                      o3_ref, om_ref):
    x = pa_ref[0] + pb_ref[0]
    y = jnp.dot(x, w_ref[...], preferred_element_type=jnp.float32)
    e3 = 1.0 / (1.0 + jnp.exp(-y))
    o3_ref[...] = e3
    om_ref[...] = (x0_ref[...] + x1_ref[...] + x2_ref[...] + e3) * 0.25


def _dense_final(partials, W, e0, e1, e2):
    return pl.pallas_call(
        _dense_final_body,
        grid=(N // _BLK,),
        in_specs=[_P_SPEC_A, _P_SPEC_B, _W_SPEC,
                  _ROW_SPEC, _ROW_SPEC, _ROW_SPEC],
        out_specs=[_ROW_SPEC, _ROW_SPEC],
        out_shape=[jax.ShapeDtypeStruct((N, D), jnp.float32),
                   jax.ShapeDtypeStruct((N, D), jnp.float32)],
    )(partials, partials, W, e0, e1, e2)


def kernel(ItemAndUserEmebddings, edge_index, A_values, W0, W1, W2):
    x = ItemAndUserEmebddings
    src = edge_index[0].reshape(NW, NCHUNK, CHUNK)
    dst = edge_index[1].reshape(NW, NCHUNK, CHUNK)
    A_values = A_values.reshape(NW, NCHUNK, CHUNK)

    p = _spmm_partials(x, src, dst, A_values)
    e1 = _dense(p, W0)
    p = _spmm_partials(e1, src, dst, A_values)
    e2 = _dense(p, W1)
    p = _spmm_partials(e2, src, dst, A_values)
    e3, mean = _dense_final(p, W2, x, e1, e2)
    return (mean, x, e1, e2, e3)


# speedup vs baseline: 2.5286x; 1.0165x over previous
import functools

import jax
import jax.numpy as jnp
from jax import lax
from jax.experimental import pallas as pl
from jax.experimental.pallas import tpu as pltpu
from jax.experimental.pallas import tpu_sc as plsc

N = 10000
E = 320000
D = 128
NC = 2
NS = 16
NW = NC * NS
LANES = 16
EDGES_PER_TILE = E // NW
CHUNK = 80
NCHUNK = EDGES_PER_TILE // CHUNK
NBUF = 4
ROWS_PER_TILE = N // NS
FULL_GROUPS = CHUNK // LANES
TAIL = CHUNK - FULL_GROUPS * LANES


def _lane_broadcast(v16, e):
    idx = jnp.full((LANES, 1), e, jnp.int32)
    dn = lax.GatherDimensionNumbers(
        offset_dims=(), collapsed_slice_dims=(0,), start_index_map=(0,))
    return lax.gather(v16, idx, dn, (1,),
                      mode=lax.GatherScatterMode.PROMISE_IN_BOUNDS)


def _spmm_partials(emb, src, dst, vals):
    mesh = plsc.VectorSubcoreMesh(
        core_axis_name="c", subcore_axis_name="s", num_cores=NC,
        num_subcores=NS)

    @functools.partial(
        pl.kernel,
        out_type=jax.ShapeDtypeStruct((NC, N, D), jnp.float32),
        mesh=mesh,
        compiler_params=pltpu.CompilerParams(use_tc_tiling_on_sc=False),
        scratch_types=[
            pltpu.VMEM_SHARED((N, D), jnp.float32),
            [pltpu.VMEM((CHUNK, D), jnp.float32)] * NBUF,
            [pltpu.VMEM((CHUNK,), jnp.int32)] * NBUF,
            [pltpu.VMEM((CHUNK,), jnp.int32)] * NBUF,
            [pltpu.VMEM((CHUNK,), jnp.float32)] * NBUF,
            [pltpu.SemaphoreType.DMA] * NBUF,
            [pltpu.SemaphoreType.DMA] * NBUF,
            [pltpu.SemaphoreType.DMA] * NBUF,
            [pltpu.SemaphoreType.DMA] * NBUF,
            pltpu.SemaphoreType.DMA,
        ],
    )
    def spmm(emb_hbm, src_hbm, dst_hbm, val_hbm, out_hbm,
             acc_sh, gbufs, sbufs, dbufs, vbufs, gsems, isems, dsems,
             ssems, zsem):
        c = lax.axis_index("c")
        s = lax.axis_index("s")
        wid = s * NC + c

        def start_src(ci, k):
            pltpu.async_copy(src_hbm.at[wid, ci], sbufs[k], isems[k])

        def wait_src(k):
            pltpu.make_async_copy(
                src_hbm.at[wid, 0], sbufs[k], isems[k]).wait()

        def start_dv(ci, k):
            pltpu.async_copy(dst_hbm.at[wid, ci], dbufs[k], dsems[k])
            pltpu.async_copy(val_hbm.at[wid, ci], vbufs[k], dsems[k])

        def start_gather(k):
            pltpu.async_copy(emb_hbm.at[sbufs[k]], gbufs[k], gsems[k])

        def wait_chunk(k):
            pltpu.make_async_copy(
                emb_hbm.at[sbufs[0]], gbufs[k], gsems[k]).wait()
            pltpu.make_async_copy(
                dst_hbm.at[wid, 0], dbufs[k], dsems[k]).wait()
            pltpu.make_async_copy(
                val_hbm.at[wid, 0], vbufs[k], dsems[k]).wait()

        def start_scatter(k):
            pltpu.make_async_copy(
                gbufs[k], acc_sh.at[dbufs[k]], ssems[k]).start(add=True)

        def wait_scatter(k):
            pltpu.make_async_copy(
                gbufs[k], acc_sh.at[dbufs[k]], ssems[k]).wait()

        def scale(k):
            gb, vb = gbufs[k], vbufs[k]

            def one(e, a16, r0):
                ae = _lane_broadcast(a16, e)
                for j in range(D // LANES):
                    sl = pl.ds(j * LANES, LANES)
                    gb[r0 + e, sl] = gb[r0 + e, sl] * ae

            def group(g, gcarry):
                a16 = vb[pl.ds(g * LANES, LANES)]
                for e in range(LANES):
                    one(e, a16, g * LANES)
                return gcarry

            lax.fori_loop(0, CHUNK // LANES, group, 0)

        for k in range(NBUF - 1):
            start_src(k, k)
            start_dv(k, k)
        for k in range(NBUF - 1):
            wait_src(k)
            start_gather(k)

        zeros16 = jnp.zeros((LANES,), jnp.float32)
        zb = gbufs[NBUF - 1]

        def zrow(r, carry):
            for j in range(D // LANES):
                zb[r, pl.ds(j * LANES, LANES)] = zeros16
            return carry

        lax.fori_loop(0, CHUNK, zrow, 0)
        base = s * ROWS_PER_TILE
        nfull = ROWS_PER_TILE // CHUNK
        rem = ROWS_PER_TILE % CHUNK
        for t in range(nfull):
            pltpu.async_copy(zb, acc_sh.at[pl.ds(base + t * CHUNK, CHUNK)],
                             zsem)
        if rem:
            pltpu.async_copy(
                zb.at[pl.ds(0, rem)],
                acc_sh.at[pl.ds(base + ROWS_PER_TILE - rem, rem)], zsem)
        for t in range(nfull):
            pltpu.make_async_copy(
                zb, acc_sh.at[pl.ds(base + t * CHUNK, CHUNK)], zsem).wait()
        if rem:
            pltpu.make_async_copy(
                zb.at[pl.ds(0, rem)],
                acc_sh.at[pl.ds(base + ROWS_PER_TILE - rem, rem)],
                zsem).wait()
        plsc.subcore_barrier()

        def ring(t, carry):
            i0 = NBUF * t
            for k in range(NBUF):
                i = i0 + k
                kp = (k - 1) % NBUF
                wait_chunk(k)

                @pl.when(i + NBUF - 1 < NCHUNK)
                def _():
                    start_src(i + NBUF - 1, kp)

                scale(k)
                start_scatter(k)

                @pl.when(i >= 1)
                def _():
                    wait_scatter(kp)

                @pl.when(i + NBUF - 1 < NCHUNK)
                def _():
                    start_dv(i + NBUF - 1, kp)
                    wait_src(kp)
                    start_gather(kp)
            return carry

        lax.fori_loop(0, NCHUNK // NBUF, ring, 0)
        wait_scatter(NBUF - 1)
        wait_chunk(0)
        scale(0)
        pltpu.sync_copy(gbufs[0], acc_sh.at[dbufs[0]], add=True)
        plsc.subcore_barrier()

        pltpu.sync_copy(
            acc_sh.at[pl.ds(s * ROWS_PER_TILE, ROWS_PER_TILE)],
            out_hbm.at[c, pl.ds(s * ROWS_PER_TILE, ROWS_PER_TILE)])

    return spmm(emb, src, dst, vals)


_BLK = 2000


_P_SPEC_A = pl.BlockSpec((1, _BLK, D), lambda i: (0, i, 0))
_P_SPEC_B = pl.BlockSpec((1, _BLK, D), lambda i: (1, i, 0))
_W_SPEC = pl.BlockSpec((D, D), lambda i: (0, 0))
_ROW_SPEC = pl.BlockSpec((_BLK, D), lambda i: (i, 0))


def _dense_body(pa_ref, pb_ref, w_ref, o_ref):
    x = pa_ref[0] + pb_ref[0]
    y = jnp.dot(x, w_ref[...], preferred_element_type=jnp.float32)
    o_ref[...] = 1.0 / (1.0 + jnp.exp(-y))


def _dense(partials, W):
    return pl.pallas_call(
        _dense_body,
        grid=(N // _BLK,),
        in_specs=[_P_SPEC_A, _P_SPEC_B, _W_SPEC],
        out_specs=_ROW_SPEC,
        out_shape=jax.ShapeDtypeStruct((N, D), jnp.float32),
    )(partials, partials, W)


def _dense_final_body(pa_ref, pb_ref, w_ref, x0_ref, x1_ref, x2_ref,
                      o3_ref, om_ref):
    x = pa_ref[0] + pb_ref[0]
    y = jnp.dot(x, w_ref[...], preferred_element_type=jnp.float32)
    e3 = 1.0 / (1.0 + jnp.exp(-y))
    o3_ref[...] = e3
    om_ref[...] = (x0_ref[...] + x1_ref[...] + x2_ref[...] + e3) * 0.25


def _dense_final(partials, W, e0, e1, e2):
    return pl.pallas_call(
        _dense_final_body,
        grid=(N // _BLK,),
        in_specs=[_P_SPEC_A, _P_SPEC_B, _W_SPEC,
                  _ROW_SPEC, _ROW_SPEC, _ROW_SPEC],
        out_specs=[_ROW_SPEC, _ROW_SPEC],
        out_shape=[jax.ShapeDtypeStruct((N, D), jnp.float32),
                   jax.ShapeDtypeStruct((N, D), jnp.float32)],
    )(partials, partials, W, e0, e1, e2)


def kernel(ItemAndUserEmebddings, edge_index, A_values, W0, W1, W2):
    x = ItemAndUserEmebddings
    src = edge_index[0].reshape(NW, NCHUNK, CHUNK)
    dst = edge_index[1].reshape(NW, NCHUNK, CHUNK)
    A_values = A_values.reshape(NW, NCHUNK, CHUNK)

    p = _spmm_partials(x, src, dst, A_values)
    e1 = _dense(p, W0)
    p = _spmm_partials(e1, src, dst, A_values)
    e2 = _dense(p, W1)
    p = _spmm_partials(e2, src, dst, A_values)
    e3, mean = _dense_final(p, W2, x, e1, e2)
    return (mean, x, e1, e2, e3)
